# Initial kernel scaffold; baseline (speedup 1.0000x reference)
import jax
import jax.numpy as jnp
from jax.experimental import pallas as pl


def kernel(x, edge_index, prev_state, Wx, Wh, bx, bh, w_peep, b_gate, W_out, b_out):
    raise NotImplementedError("write your pallas kernel here")


# speedup vs baseline: 10.0754x; 10.0754x over previous
import functools

import jax
import jax.numpy as jnp
from jax import lax
from jax.experimental import pallas as pl
from jax.experimental.pallas import tpu as pltpu
from jax.experimental.pallas import tpu_sc as plsc

NN = 10000
EE = 320000
FIN = 128
FH = 64
FOUT = 10

NC = 2
NS = 16
NP = 10240

_MESH = plsc.VectorSubcoreMesh(core_axis_name="c", subcore_axis_name="s")


EPT_H = EE // (NC * NS)


@functools.partial(
    pl.kernel,
    out_type=jax.ShapeDtypeStruct((NC * NS, 1, NP), jnp.float32),
    mesh=_MESH,
    scratch_types=[
        pltpu.VMEM((EPT_H,), jnp.int32),
        pltpu.VMEM((NP,), jnp.float32),
    ],
    compiler_params=pltpu.CompilerParams(needs_layout_passes=False),
)
def _histo_kernel(row_hbm, hist_out, idx_all, hist):
    cid = lax.axis_index("c")
    sid = lax.axis_index("s")
    wid = cid * NS + sid

    zv = jnp.zeros((16,), jnp.float32)

    def zero(i, _):
        hist[pl.ds(i * 16, 16)] = zv
        return 0

    lax.fori_loop(0, NP // 16, zero, 0)

    pltpu.sync_copy(row_hbm.at[pl.ds(wid * EPT_H, EPT_H)], idx_all)

    ones = jnp.full((16,), 1.0, jnp.float32)

    def acc(i, _):
        idx16 = idx_all[pl.ds(i * 16, 16)]
        plsc.addupdate_scatter(hist, [idx16], ones)
        return 0

    lax.fori_loop(0, EPT_H // 16, acc, 0)

    pltpu.sync_copy(hist, hist_out.at[wid, 0])


BD = 1000


def _scale_body(x_ref, h_ref, deg_ref, vp0_ref, vp1_ref):
    deg = deg_ref[...]
    pos = deg > 0.0
    dinv = jnp.where(pos, lax.rsqrt(jnp.where(pos, deg, 1.0)), 0.0)
    vp0_ref[...] = dinv * x_ref[...]
    vp1_ref[...] = dinv * h_ref[...]


def _scale(x, hs, deg):
    grid = NN // BD
    return pl.pallas_call(
        _scale_body,
        grid=(grid,),
        in_specs=[
            pl.BlockSpec((BD, FIN), lambda i: (i, 0)),
            pl.BlockSpec((BD, FH), lambda i: (i, 0)),
            pl.BlockSpec((BD, 1), lambda i: (i, 0)),
        ],
        out_specs=[
            pl.BlockSpec((BD, FIN), lambda i: (i, 0)),
            pl.BlockSpec((BD, FH), lambda i: (i, 0)),
        ],
        out_shape=[
            jax.ShapeDtypeStruct((NN, FIN), jnp.float32),
            jax.ShapeDtypeStruct((NN, FH), jnp.float32),
        ],
    )(x, hs, deg)


ER = 2500
EC = 128


def _pack_body(r_ref, c_ref, p_ref):
    p_ref[...] = r_ref[...] * 16384 + c_ref[...]


def _pack(row, col):
    r2 = row.reshape(ER, EC)
    c2 = col.reshape(ER, EC)
    return pl.pallas_call(
        _pack_body,
        out_shape=jax.ShapeDtypeStruct((ER, EC), jnp.int32),
    )(r2, c2).reshape(EE)


FT = FIN + FH
FPT = FT // (NC * NS)
SB_E = 6400
NSB_E = EE // SB_E


@functools.partial(
    pl.kernel,
    out_type=jax.ShapeDtypeStruct((FT, 1, NN), jnp.float32),
    mesh=_MESH,
    scratch_types=[
        pltpu.VMEM((SB_E,), jnp.int32)] +
        [pltpu.VMEM((NN,), jnp.float32) for _ in range(2 * FPT)],
    compiler_params=pltpu.CompilerParams(needs_layout_passes=False),
)
def _edge_kernel(packed_hbm, vpt_hbm, agg_out, pbuf, *vrows_accs):
    cid = lax.axis_index("c")
    sid = lax.axis_index("s")
    wid = cid * NS + sid

    vrows = vrows_accs[:FPT]
    accs = vrows_accs[FPT:]

    zv = jnp.zeros((16,), jnp.float32)

    def zero(i, _):
        for a in accs:
            a[pl.ds(i * 16, 16)] = zv
        return 0

    lax.fori_loop(0, NN // 16, zero, 0)

    for j, vr in enumerate(vrows):
        pltpu.sync_copy(vpt_hbm.at[wid * FPT + j, 0], vr)

    def superblock(b, _):
        pltpu.sync_copy(packed_hbm.at[pl.ds(b * SB_E, SB_E)], pbuf)

        def vstep(i, _):
            for u in range(4):
                pk = pbuf[pl.ds(i * 64 + u * 16, 16)]
                row16 = lax.shift_right_logical(pk, 14)
                col16 = jnp.bitwise_and(pk, 16383)
                gs = [plsc.load_gather(vr, [col16]) for vr in vrows]
                for a, g in zip(accs, gs):
                    plsc.addupdate_scatter(a, [row16], g)
            return 0

        lax.fori_loop(0, SB_E // 64, vstep, 0)
        return 0

    lax.fori_loop(0, NSB_E, superblock, 0)

    for j, a in enumerate(accs):
        pltpu.sync_copy(a, agg_out.at[wid * FPT + j, 0])


def _dense_body(x_ref, h_ref, c_ref, agg0_ref, agg1_ref, deg_ref,
                wx0_ref, wx1_ref, wh0_ref, wh1_ref, btot_ref, wp_ref,
                wout_ref, bout_ref, out_ref, hr_ref, cn_ref):
    deg = deg_ref[...]
    pos = deg > 0.0
    dinv = jnp.where(pos, lax.rsqrt(jnp.where(pos, deg, 1.0)), 0.0)
    sgn = jnp.where(pos, 1.0, -1.0)

    xb = x_ref[...]
    hb = h_ref[...]
    cb = c_ref[...]
    aggx = agg0_ref[...]
    aggh = agg1_ref[...]

    lx = sgn * xb - 2.0 * dinv * aggx
    lh = sgn * hb - 2.0 * dinv * aggh

    hi = jax.lax.Precision.HIGHEST
    g = (
        jnp.dot(xb, wx0_ref[...], precision=hi, preferred_element_type=jnp.float32)
        + jnp.dot(lx, wx1_ref[...], precision=hi, preferred_element_type=jnp.float32)
        + jnp.dot(hb, wh0_ref[...], precision=hi, preferred_element_type=jnp.float32)
        + jnp.dot(lh, wh1_ref[...], precision=hi, preferred_element_type=jnp.float32)
        + btot_ref[...]
    )

    ig = jax.nn.sigmoid(g[:, 0:FH] + wp_ref[0:1, :] * cb)
    fg = jax.nn.sigmoid(g[:, FH:2 * FH] + wp_ref[1:2, :] * cb)
    tg = jnp.tanh(g[:, 2 * FH:3 * FH])
    cn = fg * cb + ig * tg
    og = jax.nn.sigmoid(g[:, 3 * FH:4 * FH] + wp_ref[2:3, :] * cn)
    hn = og * jnp.tanh(cn)
    hr = jnp.maximum(hn, 0.0)

    out_ref[...] = (
        jnp.dot(hr, wout_ref[...], precision=hi, preferred_element_type=jnp.float32)
        + bout_ref[...]
    )
    hr_ref[...] = hr
    cn_ref[...] = cn


def _dense(x, hs, cs, agg0, agg1, deg, wx0, wx1, wh0, wh1, btot, wp,
           wout, bout):
    grid = NN // BD
    blk = lambda r, c: pl.BlockSpec((r, c), lambda i: (i, 0))
    full = lambda r, c: pl.BlockSpec((r, c), lambda i: (0, 0))
    return pl.pallas_call(
        _dense_body,
        grid=(grid,),
        in_specs=[
            blk(BD, FIN), blk(BD, FH), blk(BD, FH),
            blk(BD, FIN), blk(BD, FH), blk(BD, 1),
            full(FIN, 4 * FH), full(FIN, 4 * FH),
            full(FH, 4 * FH), full(FH, 4 * FH),
            full(1, 4 * FH), full(3, FH), full(FH, FOUT), full(1, FOUT),
        ],
        out_specs=[
            pl.BlockSpec((BD, FOUT), lambda i: (i, 0)),
            pl.BlockSpec((BD, FH), lambda i: (i, 0)),
            pl.BlockSpec((BD, FH), lambda i: (i, 0)),
        ],
        out_shape=[
            jax.ShapeDtypeStruct((NN, FOUT), jnp.float32),
            jax.ShapeDtypeStruct((NN, FH), jnp.float32),
            jax.ShapeDtypeStruct((NN, FH), jnp.float32),
        ],
    )(x, hs, cs, agg0, agg1, deg, wx0, wx1, wh0, wh1, btot, wp, wout,
      bout)


def kernel(x, edge_index, prev_state, Wx, Wh, bx, bh, w_peep, b_gate, W_out, b_out):
    row = edge_index[0]
    col = edge_index[1]
    hs = prev_state[0]
    cs = prev_state[1]

    hist = _histo_kernel(row)
    deg = jnp.sum(hist[:, 0, :], axis=0)[:NN, None]
    vp0, vp1 = _scale(x, hs, deg)
    vpt = jnp.transpose(jnp.concatenate([vp0, vp1], axis=1)).reshape(FT, 1, NN)
    packed = _pack(row, col)
    aggt = _edge_kernel(packed, vpt)
    agg = jnp.transpose(aggt.reshape(FT, NN))
    agg0 = agg[:, :FIN]
    agg1 = agg[:, FIN:]

    wx0 = jnp.transpose(Wx[:, 0], (1, 0, 2)).reshape(FIN, 4 * FH)
    wx1 = jnp.transpose(Wx[:, 1], (1, 0, 2)).reshape(FIN, 4 * FH)
    wh0 = jnp.transpose(Wh[:, 0], (1, 0, 2)).reshape(FH, 4 * FH)
    wh1 = jnp.transpose(Wh[:, 1], (1, 0, 2)).reshape(FH, 4 * FH)
    btot = (bx + bh + b_gate).reshape(1, 4 * FH)
    bout = b_out.reshape(1, FOUT)

    out, hr, cn = _dense(x, hs, cs, agg0, agg1, deg,
                         wx0, wx1, wh0, wh1, btot, w_peep, W_out, bout)
    return out, jnp.stack((hr, cn))
